# Initial kernel scaffold; baseline (speedup 1.0000x reference)
import jax
import jax.numpy as jnp
from jax.experimental import pallas as pl


def kernel(x, adj, W_enc1, b_enc1, W_enc2, b_enc2, W_attr1, b_attr1, W_attr2, b_attr2, W_str, b_str):
    raise NotImplementedError("write your pallas kernel here")


# speedup vs baseline: 16.5180x; 16.5180x over previous
import functools

import jax
import jax.numpy as jnp
from jax.experimental import pallas as pl
from jax.experimental.pallas import tpu as pltpu

_BF = jnp.bfloat16
_F32 = jnp.float32


def _pass1_body(adj_ref, x_ref, w_ref, b_ref, h_ref, adjb_ref, p_ref):
    @pl.when(pl.program_id(0) == 0)
    def _():
        p_ref[...] = jnp.dot(
            x_ref[...].astype(_BF), w_ref[...].astype(_BF),
            preferred_element_type=_F32).astype(_BF)

    ab = adj_ref[...].astype(_BF)
    adjb_ref[...] = ab
    acc = jnp.dot(ab, p_ref[...], preferred_element_type=_F32) + b_ref[...]
    h_ref[...] = jnp.maximum(acc, 0.0)


def _pass2_body(adjb_ref, h_ref, w_ref, b_ref, z_ref, p_ref):
    @pl.when(pl.program_id(0) == 0)
    def _():
        p_ref[...] = jnp.dot(
            h_ref[...].astype(_BF), w_ref[...].astype(_BF),
            preferred_element_type=_F32).astype(_BF)

    z_ref[...] = jnp.dot(adjb_ref[...], p_ref[...],
                         preferred_element_type=_F32) + b_ref[...]


def _pass3_body(adjb_ref, z_ref, w_ref, b_ref, h2_ref, s_ref, p_ref):
    @pl.when(pl.program_id(0) == 0)
    def _():
        p_ref[...] = jnp.dot(
            z_ref[...].astype(_BF), w_ref[...].astype(_BF),
            preferred_element_type=_F32).astype(_BF)

    acc = jnp.dot(adjb_ref[...], p_ref[...],
                  preferred_element_type=_F32) + b_ref[...]
    acc = jnp.maximum(acc, 0.0)
    h2_ref[...] = acc[:, :64]
    s_ref[...] = acc[:, 64:]


def _pass4_body(adjb_ref, h2_ref, s_ref, w_ref, b_ref, x_hat_ref, a_hat_ref,
                p_ref, sb_ref, *, rows):
    i = pl.program_id(0)

    @pl.when(i == 0)
    def _():
        p_ref[...] = jnp.dot(
            h2_ref[...].astype(_BF), w_ref[...].astype(_BF),
            preferred_element_type=_F32).astype(_BF)
        sb_ref[...] = s_ref[...].astype(_BF)

    x_hat_ref[...] = jnp.dot(adjb_ref[...], p_ref[...],
                             preferred_element_type=_F32) + b_ref[...]
    s_blk = sb_ref[pl.ds(i * rows, rows), :]
    logits = jax.lax.dot_general(
        s_blk, sb_ref[...], (((1,), (1,)), ((), ())),
        preferred_element_type=_F32)
    a_hat_ref[...] = jax.nn.sigmoid(logits)


def kernel(x, adj, W_enc1, b_enc1, W_enc2, b_enc2, W_attr1, b_attr1,
           W_attr2, b_attr2, W_str, b_str):
    n, f_in = x.shape
    h_dim = W_enc1.shape[1]

    r1 = 200
    r = 1000
    r4 = 200

    b1 = b_enc1.reshape(1, h_dim)
    b2 = b_enc2.reshape(1, h_dim)
    w_cat = jnp.concatenate([W_attr1, W_str], axis=1)
    b_cat = jnp.concatenate([b_attr1, b_str]).reshape(1, 2 * h_dim)
    b4 = b_attr2.reshape(1, f_in)

    seq = pltpu.CompilerParams(dimension_semantics=("arbitrary",))

    full = lambda shape: pl.BlockSpec(shape, lambda i: (0, 0))
    rowblk = lambda rr, cols: pl.BlockSpec((rr, cols), lambda i: (i, 0))

    h, adj_bf = pl.pallas_call(
        _pass1_body,
        grid=(n // r1,),
        in_specs=[rowblk(r1, n), full((n, f_in)), full((f_in, h_dim)),
                  full((1, h_dim))],
        out_specs=[rowblk(r1, h_dim), rowblk(r1, n)],
        out_shape=[jax.ShapeDtypeStruct((n, h_dim), _F32),
                   jax.ShapeDtypeStruct((n, n), _BF)],
        scratch_shapes=[pltpu.VMEM((n, h_dim), _BF)],
        compiler_params=seq,
    )(adj, x, W_enc1, b1)

    z = pl.pallas_call(
        _pass2_body,
        grid=(n // r,),
        in_specs=[rowblk(r, n), full((n, h_dim)), full((h_dim, h_dim)),
                  full((1, h_dim))],
        out_specs=rowblk(r, h_dim),
        out_shape=jax.ShapeDtypeStruct((n, h_dim), _F32),
        scratch_shapes=[pltpu.VMEM((n, h_dim), _BF)],
        compiler_params=seq,
    )(adj_bf, h, W_enc2, b2)

    h2, s = pl.pallas_call(
        _pass3_body,
        grid=(n // r,),
        in_specs=[rowblk(r, n), full((n, h_dim)), full((h_dim, 2 * h_dim)),
                  full((1, 2 * h_dim))],
        out_specs=[rowblk(r, h_dim), rowblk(r, h_dim)],
        out_shape=[jax.ShapeDtypeStruct((n, h_dim), _F32),
                   jax.ShapeDtypeStruct((n, h_dim), _F32)],
        scratch_shapes=[pltpu.VMEM((n, 2 * h_dim), _BF)],
        compiler_params=seq,
    )(adj_bf, z, w_cat, b_cat)

    x_hat, a_hat = pl.pallas_call(
        functools.partial(_pass4_body, rows=r4),
        grid=(n // r4,),
        in_specs=[rowblk(r4, n), full((n, h_dim)), full((n, h_dim)),
                  full((h_dim, f_in)), full((1, f_in))],
        out_specs=[rowblk(r4, f_in), rowblk(r4, n)],
        out_shape=[jax.ShapeDtypeStruct((n, f_in), _F32),
                   jax.ShapeDtypeStruct((n, n), _F32)],
        scratch_shapes=[pltpu.VMEM((n, f_in), _BF),
                        pltpu.VMEM((n, h_dim), _BF)],
        compiler_params=seq,
    )(adj_bf, h2, s, W_attr2, b4)

    return (a_hat, x_hat, z)


# speedup vs baseline: 17.1856x; 1.0404x over previous
import functools

import jax
import jax.numpy as jnp
from jax.experimental import pallas as pl
from jax.experimental.pallas import tpu as pltpu

_BF = jnp.bfloat16
_F32 = jnp.float32


def _pass1_body(adj_ref, x_ref, w_ref, b_ref, h_ref, adjb_ref, p_ref):
    @pl.when(pl.program_id(0) == 0)
    def _():
        p_ref[...] = jnp.dot(
            x_ref[...].astype(_BF), w_ref[...].astype(_BF),
            preferred_element_type=_F32).astype(_BF)

    ab = adj_ref[...].astype(_BF)
    adjb_ref[...] = ab
    acc = jnp.dot(ab, p_ref[...], preferred_element_type=_F32) + b_ref[...]
    h_ref[...] = jnp.maximum(acc, 0.0)


def _pass2_body(adjb_ref, h_ref, w_ref, b_ref, z_ref, p_ref):
    @pl.when(pl.program_id(0) == 0)
    def _():
        p_ref[...] = jnp.dot(
            h_ref[...].astype(_BF), w_ref[...].astype(_BF),
            preferred_element_type=_F32).astype(_BF)

    z_ref[...] = jnp.dot(adjb_ref[...], p_ref[...],
                         preferred_element_type=_F32) + b_ref[...]


def _pass3_body(adjb_ref, z_ref, w_ref, b_ref, h2_ref, s_ref, p_ref):
    @pl.when(pl.program_id(0) == 0)
    def _():
        p_ref[...] = jnp.dot(
            z_ref[...].astype(_BF), w_ref[...].astype(_BF),
            preferred_element_type=_F32).astype(_BF)

    acc = jnp.dot(adjb_ref[...], p_ref[...],
                  preferred_element_type=_F32) + b_ref[...]
    acc = jnp.maximum(acc, 0.0)
    h2_ref[...] = acc[:, :64]
    s_ref[...] = acc[:, 64:].astype(_BF)


def _pass4_body(adjb_ref, h2_ref, s_ref, sblk_ref, w_ref, b_ref,
                x_hat_ref, a_hat_ref, p_ref, st_ref):
    @pl.when(pl.program_id(0) == 0)
    def _():
        p_ref[...] = jnp.dot(
            h2_ref[...].astype(_BF), w_ref[...].astype(_BF),
            preferred_element_type=_F32).astype(_BF)
        st_ref[...] = s_ref[...].T

    x_hat_ref[...] = jnp.dot(adjb_ref[...], p_ref[...],
                             preferred_element_type=_F32) + b_ref[...]
    logits = jnp.dot(sblk_ref[...], st_ref[...],
                     preferred_element_type=_F32)
    a_hat_ref[...] = 0.5 + 0.5 * jnp.tanh(0.5 * logits)


def kernel(x, adj, W_enc1, b_enc1, W_enc2, b_enc2, W_attr1, b_attr1,
           W_attr2, b_attr2, W_str, b_str):
    n, f_in = x.shape
    h_dim = W_enc1.shape[1]

    r1 = 200
    r = 1000
    r4 = 200

    b1 = b_enc1.reshape(1, h_dim)
    b2 = b_enc2.reshape(1, h_dim)
    w_cat = jnp.concatenate([W_attr1, W_str], axis=1)
    b_cat = jnp.concatenate([b_attr1, b_str]).reshape(1, 2 * h_dim)
    b4 = b_attr2.reshape(1, f_in)

    seq = pltpu.CompilerParams(dimension_semantics=("arbitrary",))

    full = lambda shape: pl.BlockSpec(shape, lambda i: (0, 0))
    rowblk = lambda rr, cols: pl.BlockSpec((rr, cols), lambda i: (i, 0))

    h, adj_bf = pl.pallas_call(
        _pass1_body,
        grid=(n // r1,),
        in_specs=[rowblk(r1, n), full((n, f_in)), full((f_in, h_dim)),
                  full((1, h_dim))],
        out_specs=[rowblk(r1, h_dim), rowblk(r1, n)],
        out_shape=[jax.ShapeDtypeStruct((n, h_dim), _F32),
                   jax.ShapeDtypeStruct((n, n), _BF)],
        scratch_shapes=[pltpu.VMEM((n, h_dim), _BF)],
        compiler_params=seq,
    )(adj, x, W_enc1, b1)

    z = pl.pallas_call(
        _pass2_body,
        grid=(n // r,),
        in_specs=[rowblk(r, n), full((n, h_dim)), full((h_dim, h_dim)),
                  full((1, h_dim))],
        out_specs=rowblk(r, h_dim),
        out_shape=jax.ShapeDtypeStruct((n, h_dim), _F32),
        scratch_shapes=[pltpu.VMEM((n, h_dim), _BF)],
        compiler_params=seq,
    )(adj_bf, h, W_enc2, b2)

    h2, s_bf = pl.pallas_call(
        _pass3_body,
        grid=(n // r,),
        in_specs=[rowblk(r, n), full((n, h_dim)), full((h_dim, 2 * h_dim)),
                  full((1, 2 * h_dim))],
        out_specs=[rowblk(r, h_dim), rowblk(r, h_dim)],
        out_shape=[jax.ShapeDtypeStruct((n, h_dim), _F32),
                   jax.ShapeDtypeStruct((n, h_dim), _BF)],
        scratch_shapes=[pltpu.VMEM((n, 2 * h_dim), _BF)],
        compiler_params=seq,
    )(adj_bf, z, w_cat, b_cat)

    x_hat, a_hat = pl.pallas_call(
        _pass4_body,
        grid=(n // r4,),
        in_specs=[rowblk(r4, n), full((n, h_dim)), full((n, h_dim)),
                  rowblk(r4, h_dim), full((h_dim, f_in)), full((1, f_in))],
        out_specs=[rowblk(r4, f_in), rowblk(r4, n)],
        out_shape=[jax.ShapeDtypeStruct((n, f_in), _F32),
                   jax.ShapeDtypeStruct((n, n), _F32)],
        scratch_shapes=[pltpu.VMEM((n, f_in), _BF),
                        pltpu.VMEM((h_dim, n), _BF)],
        compiler_params=seq,
    )(adj_bf, h2, s_bf, s_bf, W_attr2, b4)

    return (a_hat, x_hat, z)
